# baseline (speedup 1.0000x reference)
import functools

import jax
import jax.numpy as jnp
from jax import lax
from jax.experimental import pallas as pl
from jax.experimental.pallas import tpu as pltpu
from jax.experimental.pallas import tpu_sc as plsc

N = 10000
F = 128
H = 128
T = 4
NC = 2
NS = 16
NW = NC * NS
CHUNK = 64
NCHUNK = 160
NQ = 4
QC = NCHUNK // NQ
NBUF = 4
EW = CHUNK * NCHUNK
E_PAD = EW * NW
R = 632
N_PAD = R * NS
BLK = 1264
GRID = N_PAD // BLK

_MESH = plsc.VectorSubcoreMesh(
    core_axis_name="c", subcore_axis_name="s", num_cores=NC, num_subcores=NS)


def _hist_body(idx_hbm, ones_hbm, zeros_hbm, out_hbm, idx_v, ones_v, acc):
  c = lax.axis_index("c")
  s = lax.axis_index("s")
  pltpu.sync_copy(ones_hbm, ones_v)
  pltpu.sync_copy(zeros_hbm.at[pl.ds(s * R, R)], acc.at[pl.ds(s * R, R)])
  plsc.subcore_barrier()
  for q in range(NQ):
    pltpu.sync_copy(idx_hbm.at[c, s, q], idx_v)

    def body(j, carry):
      pltpu.sync_copy(ones_v, acc.at[idx_v.at[QC + j]], add=True)
      return carry

    lax.fori_loop(0, QC, body, 0)
  plsc.subcore_barrier()
  pltpu.sync_copy(acc.at[pl.ds(s * R, R)], out_hbm.at[c, pl.ds(s * R, R)])


_hist_call = pl.kernel(
    _hist_body,
    out_type=jax.ShapeDtypeStruct((NC, N_PAD, F), jnp.float32),
    mesh=_MESH,
    scratch_types=[
        pltpu.VMEM((2 * QC, CHUNK), jnp.int32),
        pltpu.VMEM((CHUNK, F), jnp.float32),
        pltpu.VMEM_SHARED((N_PAD, F), jnp.float32),
    ],
)


def _make_prop(nt):
  def body(table_hbm, idx_hbm, zeros_hbm, out_hbm,
           idx_v, buf0, buf1, buf2, buf3, acc,
           sg0, sg1, sg2, sg3, ss0, ss1, ss2, ss3):
    c = lax.axis_index("c")
    s = lax.axis_index("s")
    bufs = (buf0, buf1, buf2, buf3)
    sg = (sg0, sg1, sg2, sg3)
    ss = (ss0, ss1, ss2, ss3)
    for t in range(nt):
      table = table_hbm.at[t]
      pltpu.sync_copy(zeros_hbm.at[pl.ds(s * R, R)], acc.at[pl.ds(s * R, R)])
      plsc.subcore_barrier()
      for q in range(NQ):
        pltpu.sync_copy(idx_hbm.at[c, s, q], idx_v)
        pltpu.async_copy(table.at[idx_v.at[0]], buf0, sg0)
        pltpu.async_copy(table.at[idx_v.at[1]], buf1, sg1)

        def body2(j2, carry):
          for p in range(NBUF):
            j = j2 * NBUF + p
            pltpu.make_async_copy(table.at[idx_v.at[j]], bufs[p], sg[p]).wait()
            pltpu.async_copy(bufs[p], acc.at[idx_v.at[QC + j]], ss[p], add=True)
            np_ = (p + 2) % NBUF

            @pl.when(j + 2 < QC)
            def _issue():
              @pl.when(j >= 2)
              def _retire():
                pltpu.make_async_copy(
                    bufs[np_], acc.at[idx_v.at[QC]], ss[np_]).wait()

              pltpu.async_copy(table.at[idx_v.at[j + 2]], bufs[np_], sg[np_])
          return carry

        lax.fori_loop(0, QC // NBUF, body2, 0)
        for k in range(QC - NBUF, QC):
          p = k % NBUF
          pltpu.make_async_copy(bufs[p], acc.at[idx_v.at[QC]], ss[p]).wait()
      plsc.subcore_barrier()
      pltpu.sync_copy(acc.at[pl.ds(s * R, R)], out_hbm.at[t, c, pl.ds(s * R, R)])

  return pl.kernel(
      body,
      out_type=jax.ShapeDtypeStruct((nt, NC, N_PAD, F), jnp.float32),
      mesh=_MESH,
      scratch_types=[
          pltpu.VMEM((2 * QC, CHUNK), jnp.int32),
          pltpu.VMEM((CHUNK, F), jnp.float32),
          pltpu.VMEM((CHUNK, F), jnp.float32),
          pltpu.VMEM((CHUNK, F), jnp.float32),
          pltpu.VMEM((CHUNK, F), jnp.float32),
          pltpu.VMEM_SHARED((N_PAD, F), jnp.float32),
          pltpu.SemaphoreType.DMA,
          pltpu.SemaphoreType.DMA,
          pltpu.SemaphoreType.DMA,
          pltpu.SemaphoreType.DMA,
          pltpu.SemaphoreType.DMA,
          pltpu.SemaphoreType.DMA,
          pltpu.SemaphoreType.DMA,
          pltpu.SemaphoreType.DMA,
      ],
  )


_prop1 = _make_prop(1)
_prop4 = _make_prop(T)


def _prep_body(hist_ref, x_ref, dis_ref, ux_ref):
  deg = hist_ref[0, :, 0:1] + hist_ref[1, :, 0:1] + 1.0
  d = lax.rsqrt(deg)
  dis_ref[...] = jnp.broadcast_to(d, (BLK, F))
  for t in range(T):
    ux_ref[t] = x_ref[t] * d


_prep_call = pl.pallas_call(
    _prep_body,
    grid=(GRID,),
    in_specs=[
        pl.BlockSpec((NC, BLK, F), lambda i: (0, i, 0)),
        pl.BlockSpec((T, BLK, F), lambda i: (0, i, 0)),
    ],
    out_specs=[
        pl.BlockSpec((BLK, F), lambda i: (i, 0)),
        pl.BlockSpec((T, BLK, F), lambda i: (0, i, 0)),
    ],
    out_shape=[
        jax.ShapeDtypeStruct((N_PAD, F), jnp.float32),
        jax.ShapeDtypeStruct((T, N_PAD, F), jnp.float32),
    ],
)


def _make_cell(has_h):
  def body(*refs):
    if has_h:
      (sx_ref, ux_ref, sh_ref, uh_ref, c_ref, dis_ref, wx_ref, wh_ref, b_ref,
       h_o, c_o, uh_o) = refs
    else:
      (sx_ref, ux_ref, c_ref, dis_ref, wx_ref, b_ref, h_o, c_o, uh_o) = refs
    dis = dis_ref[...]
    px = dis * (sx_ref[0] + sx_ref[1] + ux_ref[...])
    cc = jnp.dot(px, wx_ref[...], preferred_element_type=jnp.float32)
    cc = cc + b_ref[...]
    if has_h:
      ph = dis * (sh_ref[0] + sh_ref[1] + uh_ref[...])
      cc = cc + jnp.dot(ph, wh_ref[...], preferred_element_type=jnp.float32)
    gi = jax.nn.sigmoid(cc[:, 0:H])
    gf = jax.nn.sigmoid(cc[:, H:2 * H])
    go = jax.nn.sigmoid(cc[:, 2 * H:3 * H])
    gg = jnp.tanh(cc[:, 3 * H:4 * H])
    c_new = gf * c_ref[...] + gi * gg
    h_new = go * jnp.tanh(c_new)
    h_o[...] = h_new
    c_o[...] = c_new
    uh_o[...] = dis * h_new

  part = pl.BlockSpec((NC, BLK, F), lambda i: (0, i, 0))
  full = pl.BlockSpec((BLK, F), lambda i: (i, 0))
  wspec = pl.BlockSpec((F, 4 * H), lambda i: (0, 0))
  bspec = pl.BlockSpec((1, 4 * H), lambda i: (0, 0))
  if has_h:
    in_specs = [part, full, part, full, full, full, wspec, wspec, bspec]
  else:
    in_specs = [part, full, full, full, wspec, bspec]
  return pl.pallas_call(
      body,
      grid=(GRID,),
      in_specs=in_specs,
      out_specs=[full, full, full],
      out_shape=[jax.ShapeDtypeStruct((N_PAD, F), jnp.float32)] * 3,
  )


_cell_h = _make_cell(True)
_cell_nh = _make_cell(False)


def kernel(x, edge_index, W0, b0, W1, b1):
  src = edge_index[0]
  dst = edge_index[1]
  pad = jnp.full((E_PAD - src.shape[0],), N, dtype=jnp.int32)
  src_q = jnp.concatenate([src, pad]).reshape(NC, NS, NQ, QC, CHUNK)
  dst_q = jnp.concatenate([dst, pad]).reshape(NC, NS, NQ, QC, CHUNK)
  idx_all = jnp.concatenate([src_q, dst_q], axis=3)

  zeros128 = jnp.zeros((N_PAD, F), jnp.float32)
  ones128 = jnp.ones((CHUNK, F), jnp.float32)

  hist = _hist_call(idx_all, ones128, zeros128)
  x_pad = jnp.pad(x[0], ((0, 0), (0, N_PAD - N), (0, 0)))
  dis, ux0 = _prep_call(hist, x_pad)

  b0r = b0.reshape(1, 4 * H)
  b1r = b1.reshape(1, 4 * H)
  params = [(W0[:F], W0[F:], b0r), (W1[:H], W1[H:], b1r)]

  ux = ux0
  h = c = None
  for layer in range(2):
    wx, wh, br = params[layer]
    sx_all = _prop4(ux, idx_all, zeros128)
    outs = []
    for t in range(T):
      if t == 0:
        c_prev = zeros128
        h, c, uh = _cell_nh(sx_all[t], ux[t], c_prev, dis, wx, br)
      else:
        sh = _prop1(uh[None], idx_all, zeros128)
        h, c, uh = _cell_h(sx_all[t], ux[t], sh[0], uh, c, dis, wx, wh, br)
      outs.append(uh)
    ux = jnp.stack(outs)

  return (h[:N][None], c[:N][None])


# speedup vs baseline: 1.0374x; 1.0374x over previous
import functools

import jax
import jax.numpy as jnp
from jax import lax
from jax.experimental import pallas as pl
from jax.experimental.pallas import tpu as pltpu
from jax.experimental.pallas import tpu_sc as plsc

N = 10000
F = 128
H = 128
T = 4
NC = 2
NS = 16
NW = NC * NS
CHUNK = 64
NCHUNK = 160
NQ = 8
QC = NCHUNK // NQ
NBUF = 5
EW = CHUNK * NCHUNK
E_PAD = EW * NW
R = 632
N_PAD = R * NS
BLK = 1264
GRID = N_PAD // BLK

_MESH = plsc.VectorSubcoreMesh(
    core_axis_name="c", subcore_axis_name="s", num_cores=NC, num_subcores=NS)


def _hist_body(idx_hbm, ones_hbm, zeros_hbm, out_hbm, idx_v, ones_v, acc):
  c = lax.axis_index("c")
  s = lax.axis_index("s")
  pltpu.sync_copy(ones_hbm, ones_v)
  pltpu.sync_copy(zeros_hbm.at[pl.ds(s * R, R)], acc.at[pl.ds(s * R, R)])
  plsc.subcore_barrier()
  for q in range(NQ):
    pltpu.sync_copy(idx_hbm.at[c, s, q], idx_v)

    def body(j, carry):
      pltpu.sync_copy(ones_v, acc.at[idx_v.at[QC + j]], add=True)
      return carry

    lax.fori_loop(0, QC, body, 0)
  plsc.subcore_barrier()
  pltpu.sync_copy(acc.at[pl.ds(s * R, R)], out_hbm.at[c, pl.ds(s * R, R)])


_hist_call = pl.kernel(
    _hist_body,
    out_type=jax.ShapeDtypeStruct((NC, N_PAD, F), jnp.float32),
    mesh=_MESH,
    scratch_types=[
        pltpu.VMEM((2 * QC, CHUNK), jnp.int32),
        pltpu.VMEM((CHUNK, F), jnp.float32),
        pltpu.VMEM_SHARED((N_PAD, F), jnp.float32),
    ],
)


def _make_prop(nt):
  def body(table_hbm, idx_hbm, zeros_hbm, out_hbm,
           idx_v, buf0, buf1, buf2, buf3, buf4, acc,
           sg0, sg1, sg2, sg3, sg4, ss0, ss1, ss2, ss3, ss4):
    c = lax.axis_index("c")
    s = lax.axis_index("s")
    bufs = (buf0, buf1, buf2, buf3, buf4)
    sg = (sg0, sg1, sg2, sg3, sg4)
    ss = (ss0, ss1, ss2, ss3, ss4)
    for t in range(nt):
      table = table_hbm.at[t]
      pltpu.sync_copy(zeros_hbm.at[pl.ds(s * R, R)], acc.at[pl.ds(s * R, R)])
      plsc.subcore_barrier()
      for q in range(NQ):
        pltpu.sync_copy(idx_hbm.at[c, s, q], idx_v)
        pltpu.async_copy(table.at[idx_v.at[0]], buf0, sg0)
        pltpu.async_copy(table.at[idx_v.at[1]], buf1, sg1)

        def body2(j2, carry):
          for p in range(NBUF):
            j = j2 * NBUF + p
            pltpu.make_async_copy(table.at[idx_v.at[j]], bufs[p], sg[p]).wait()
            pltpu.async_copy(bufs[p], acc.at[idx_v.at[QC + j]], ss[p], add=True)
            np_ = (p + 2) % NBUF

            @pl.when(j + 2 < QC)
            def _issue():
              @pl.when(j >= 3)
              def _retire():
                pltpu.make_async_copy(
                    bufs[np_], acc.at[idx_v.at[QC]], ss[np_]).wait()

              pltpu.async_copy(table.at[idx_v.at[j + 2]], bufs[np_], sg[np_])
          return carry

        lax.fori_loop(0, QC // NBUF, body2, 0)
        for k in range(QC - NBUF, QC):
          p = k % NBUF
          pltpu.make_async_copy(bufs[p], acc.at[idx_v.at[QC]], ss[p]).wait()
      plsc.subcore_barrier()
      pltpu.sync_copy(acc.at[pl.ds(s * R, R)], out_hbm.at[t, c, pl.ds(s * R, R)])

  return pl.kernel(
      body,
      out_type=jax.ShapeDtypeStruct((nt, NC, N_PAD, F), jnp.float32),
      mesh=_MESH,
      scratch_types=(
          [pltpu.VMEM((2 * QC, CHUNK), jnp.int32)]
          + [pltpu.VMEM((CHUNK, F), jnp.float32)] * NBUF
          + [pltpu.VMEM_SHARED((N_PAD, F), jnp.float32)]
          + [pltpu.SemaphoreType.DMA] * (2 * NBUF)
      ),
  )


_prop1 = _make_prop(1)
_prop4 = _make_prop(T)


def _prep_body(hist_ref, x_ref, dis_ref, ux_ref):
  deg = hist_ref[0, :, 0:1] + hist_ref[1, :, 0:1] + 1.0
  d = lax.rsqrt(deg)
  dis_ref[...] = jnp.broadcast_to(d, (BLK, F))
  for t in range(T):
    ux_ref[t] = x_ref[t] * d


_prep_call = pl.pallas_call(
    _prep_body,
    grid=(GRID,),
    in_specs=[
        pl.BlockSpec((NC, BLK, F), lambda i: (0, i, 0)),
        pl.BlockSpec((T, BLK, F), lambda i: (0, i, 0)),
    ],
    out_specs=[
        pl.BlockSpec((BLK, F), lambda i: (i, 0)),
        pl.BlockSpec((T, BLK, F), lambda i: (0, i, 0)),
    ],
    out_shape=[
        jax.ShapeDtypeStruct((N_PAD, F), jnp.float32),
        jax.ShapeDtypeStruct((T, N_PAD, F), jnp.float32),
    ],
)


def _make_cell(has_h):
  def body(*refs):
    if has_h:
      (sx_ref, ux_ref, sh_ref, uh_ref, c_ref, dis_ref, wx_ref, wh_ref, b_ref,
       h_o, c_o, uh_o) = refs
    else:
      (sx_ref, ux_ref, c_ref, dis_ref, wx_ref, b_ref, h_o, c_o, uh_o) = refs
    dis = dis_ref[...]
    px = dis * (sx_ref[0] + sx_ref[1] + ux_ref[...])
    cc = jnp.dot(px, wx_ref[...], preferred_element_type=jnp.float32)
    cc = cc + b_ref[...]
    if has_h:
      ph = dis * (sh_ref[0] + sh_ref[1] + uh_ref[...])
      cc = cc + jnp.dot(ph, wh_ref[...], preferred_element_type=jnp.float32)
    gi = jax.nn.sigmoid(cc[:, 0:H])
    gf = jax.nn.sigmoid(cc[:, H:2 * H])
    go = jax.nn.sigmoid(cc[:, 2 * H:3 * H])
    gg = jnp.tanh(cc[:, 3 * H:4 * H])
    c_new = gf * c_ref[...] + gi * gg
    h_new = go * jnp.tanh(c_new)
    h_o[...] = h_new
    c_o[...] = c_new
    uh_o[...] = dis * h_new

  part = pl.BlockSpec((NC, BLK, F), lambda i: (0, i, 0))
  full = pl.BlockSpec((BLK, F), lambda i: (i, 0))
  wspec = pl.BlockSpec((F, 4 * H), lambda i: (0, 0))
  bspec = pl.BlockSpec((1, 4 * H), lambda i: (0, 0))
  if has_h:
    in_specs = [part, full, part, full, full, full, wspec, wspec, bspec]
  else:
    in_specs = [part, full, full, full, wspec, bspec]
  return pl.pallas_call(
      body,
      grid=(GRID,),
      in_specs=in_specs,
      out_specs=[full, full, full],
      out_shape=[jax.ShapeDtypeStruct((N_PAD, F), jnp.float32)] * 3,
  )


_cell_h = _make_cell(True)
_cell_nh = _make_cell(False)


def kernel(x, edge_index, W0, b0, W1, b1):
  src = edge_index[0]
  dst = edge_index[1]
  pad = jnp.full((E_PAD - src.shape[0],), N, dtype=jnp.int32)
  src_q = jnp.concatenate([src, pad]).reshape(NC, NS, NQ, QC, CHUNK)
  dst_q = jnp.concatenate([dst, pad]).reshape(NC, NS, NQ, QC, CHUNK)
  idx_all = jnp.concatenate([src_q, dst_q], axis=3)

  zeros128 = jnp.zeros((N_PAD, F), jnp.float32)
  ones128 = jnp.ones((CHUNK, F), jnp.float32)

  hist = _hist_call(idx_all, ones128, zeros128)
  x_pad = jnp.pad(x[0], ((0, 0), (0, N_PAD - N), (0, 0)))
  dis, ux0 = _prep_call(hist, x_pad)

  b0r = b0.reshape(1, 4 * H)
  b1r = b1.reshape(1, 4 * H)
  params = [(W0[:F], W0[F:], b0r), (W1[:H], W1[H:], b1r)]

  ux = ux0
  h = c = None
  for layer in range(2):
    wx, wh, br = params[layer]
    sx_all = _prop4(ux, idx_all, zeros128)
    outs = []
    for t in range(T):
      if t == 0:
        c_prev = zeros128
        h, c, uh = _cell_nh(sx_all[t], ux[t], c_prev, dis, wx, br)
      else:
        sh = _prop1(uh[None], idx_all, zeros128)
        h, c, uh = _cell_h(sx_all[t], ux[t], sh[0], uh, c, dis, wx, wh, br)
      outs.append(uh)
    ux = jnp.stack(outs)

  return (h[:N][None], c[:N][None])


# speedup vs baseline: 1.0415x; 1.0040x over previous
import functools

import jax
import jax.numpy as jnp
from jax import lax
from jax.experimental import pallas as pl
from jax.experimental.pallas import tpu as pltpu
from jax.experimental.pallas import tpu_sc as plsc

N = 10000
F = 128
H = 128
T = 4
NC = 2
NS = 16
NW = NC * NS
CHUNK = 128
NCHUNK = 80
NQ = 2
QC = NCHUNK // NQ
NBUF = 2
EW = CHUNK * NCHUNK
E_PAD = EW * NW
R = 632
N_PAD = R * NS
BLK = 1264
GRID = N_PAD // BLK

_MESH = plsc.VectorSubcoreMesh(
    core_axis_name="c", subcore_axis_name="s", num_cores=NC, num_subcores=NS)


def _hist_body(idx_hbm, ones_hbm, zeros_hbm, out_hbm, idx_v, ones_v, acc):
  c = lax.axis_index("c")
  s = lax.axis_index("s")
  pltpu.sync_copy(ones_hbm, ones_v)
  pltpu.sync_copy(zeros_hbm.at[pl.ds(s * R, R)], acc.at[pl.ds(s * R, R)])
  plsc.subcore_barrier()
  for q in range(NQ):
    pltpu.sync_copy(idx_hbm.at[c, s, q], idx_v)

    def body(j, carry):
      pltpu.sync_copy(ones_v, acc.at[idx_v.at[QC + j]], add=True)
      return carry

    lax.fori_loop(0, QC, body, 0)
  plsc.subcore_barrier()
  pltpu.sync_copy(acc.at[pl.ds(s * R, R)], out_hbm.at[c, pl.ds(s * R, R)])


_hist_call = pl.kernel(
    _hist_body,
    out_type=jax.ShapeDtypeStruct((NC, N_PAD, F), jnp.float32),
    mesh=_MESH,
    scratch_types=[
        pltpu.VMEM((2 * QC, CHUNK), jnp.int32),
        pltpu.VMEM((CHUNK, F), jnp.float32),
        pltpu.VMEM_SHARED((N_PAD, F), jnp.float32),
    ],
)


def _make_prop(nt):
  def body(table_hbm, idx_hbm, zeros_hbm, out_hbm,
           idx_v, buf0, buf1, acc, sg0, sg1):
    c = lax.axis_index("c")
    s = lax.axis_index("s")
    bufs = (buf0, buf1)
    sg = (sg0, sg1)
    for t in range(nt):
      table = table_hbm.at[t]
      pltpu.sync_copy(zeros_hbm.at[pl.ds(s * R, R)], acc.at[pl.ds(s * R, R)])
      plsc.subcore_barrier()
      for q in range(NQ):
        pltpu.sync_copy(idx_hbm.at[c, s, q], idx_v)
        pltpu.async_copy(table.at[idx_v.at[0]], buf0, sg0)
        pltpu.async_copy(table.at[idx_v.at[1]], buf1, sg1)

        def body2(j2, carry):
          for p in range(NBUF):
            j = j2 * NBUF + p
            pltpu.make_async_copy(table.at[idx_v.at[j]], bufs[p], sg[p]).wait()
            pltpu.sync_copy(bufs[p], acc.at[idx_v.at[QC + j]], add=True)

            @pl.when(j + 2 < QC)
            def _issue():
              pltpu.async_copy(table.at[idx_v.at[j + 2]], bufs[p], sg[p])
          return carry

        lax.fori_loop(0, QC // NBUF, body2, 0)
      plsc.subcore_barrier()
      pltpu.sync_copy(acc.at[pl.ds(s * R, R)], out_hbm.at[t, c, pl.ds(s * R, R)])

  return pl.kernel(
      body,
      out_type=jax.ShapeDtypeStruct((nt, NC, N_PAD, F), jnp.float32),
      mesh=_MESH,
      scratch_types=(
          [pltpu.VMEM((2 * QC, CHUNK), jnp.int32)]
          + [pltpu.VMEM((CHUNK, F), jnp.float32)] * NBUF
          + [pltpu.VMEM_SHARED((N_PAD, F), jnp.float32)]
          + [pltpu.SemaphoreType.DMA] * NBUF
      ),
  )


_prop1 = _make_prop(1)
_prop4 = _make_prop(T)


def _prep_body(hist_ref, x_ref, dis_ref, ux_ref):
  deg = hist_ref[0, :, 0:1] + hist_ref[1, :, 0:1] + 1.0
  d = lax.rsqrt(deg)
  dis_ref[...] = jnp.broadcast_to(d, (BLK, F))
  for t in range(T):
    ux_ref[t] = x_ref[t] * d


_prep_call = pl.pallas_call(
    _prep_body,
    grid=(GRID,),
    in_specs=[
        pl.BlockSpec((NC, BLK, F), lambda i: (0, i, 0)),
        pl.BlockSpec((T, BLK, F), lambda i: (0, i, 0)),
    ],
    out_specs=[
        pl.BlockSpec((BLK, F), lambda i: (i, 0)),
        pl.BlockSpec((T, BLK, F), lambda i: (0, i, 0)),
    ],
    out_shape=[
        jax.ShapeDtypeStruct((N_PAD, F), jnp.float32),
        jax.ShapeDtypeStruct((T, N_PAD, F), jnp.float32),
    ],
)


def _make_cell(has_h):
  def body(*refs):
    if has_h:
      (sx_ref, ux_ref, sh_ref, uh_ref, c_ref, dis_ref, wx_ref, wh_ref, b_ref,
       h_o, c_o, uh_o) = refs
    else:
      (sx_ref, ux_ref, c_ref, dis_ref, wx_ref, b_ref, h_o, c_o, uh_o) = refs
    dis = dis_ref[...]
    px = dis * (sx_ref[0] + sx_ref[1] + ux_ref[...])
    cc = jnp.dot(px, wx_ref[...], preferred_element_type=jnp.float32)
    cc = cc + b_ref[...]
    if has_h:
      ph = dis * (sh_ref[0] + sh_ref[1] + uh_ref[...])
      cc = cc + jnp.dot(ph, wh_ref[...], preferred_element_type=jnp.float32)
    gi = jax.nn.sigmoid(cc[:, 0:H])
    gf = jax.nn.sigmoid(cc[:, H:2 * H])
    go = jax.nn.sigmoid(cc[:, 2 * H:3 * H])
    gg = jnp.tanh(cc[:, 3 * H:4 * H])
    c_new = gf * c_ref[...] + gi * gg
    h_new = go * jnp.tanh(c_new)
    h_o[...] = h_new
    c_o[...] = c_new
    uh_o[...] = dis * h_new

  part = pl.BlockSpec((NC, BLK, F), lambda i: (0, i, 0))
  full = pl.BlockSpec((BLK, F), lambda i: (i, 0))
  wspec = pl.BlockSpec((F, 4 * H), lambda i: (0, 0))
  bspec = pl.BlockSpec((1, 4 * H), lambda i: (0, 0))
  if has_h:
    in_specs = [part, full, part, full, full, full, wspec, wspec, bspec]
  else:
    in_specs = [part, full, full, full, wspec, bspec]
  return pl.pallas_call(
      body,
      grid=(GRID,),
      in_specs=in_specs,
      out_specs=[full, full, full],
      out_shape=[jax.ShapeDtypeStruct((N_PAD, F), jnp.float32)] * 3,
  )


_cell_h = _make_cell(True)
_cell_nh = _make_cell(False)


def kernel(x, edge_index, W0, b0, W1, b1):
  src = edge_index[0]
  dst = edge_index[1]
  pad = jnp.full((E_PAD - src.shape[0],), N, dtype=jnp.int32)
  src_q = jnp.concatenate([src, pad]).reshape(NC, NS, NQ, QC, CHUNK)
  dst_q = jnp.concatenate([dst, pad]).reshape(NC, NS, NQ, QC, CHUNK)
  idx_all = jnp.concatenate([src_q, dst_q], axis=3)

  zeros128 = jnp.zeros((N_PAD, F), jnp.float32)
  ones128 = jnp.ones((CHUNK, F), jnp.float32)

  hist = _hist_call(idx_all, ones128, zeros128)
  x_pad = jnp.pad(x[0], ((0, 0), (0, N_PAD - N), (0, 0)))
  dis, ux0 = _prep_call(hist, x_pad)

  b0r = b0.reshape(1, 4 * H)
  b1r = b1.reshape(1, 4 * H)
  params = [(W0[:F], W0[F:], b0r), (W1[:H], W1[H:], b1r)]

  ux = ux0
  h = c = None
  for layer in range(2):
    wx, wh, br = params[layer]
    sx_all = _prop4(ux, idx_all, zeros128)
    outs = []
    for t in range(T):
      if t == 0:
        c_prev = zeros128
        h, c, uh = _cell_nh(sx_all[t], ux[t], c_prev, dis, wx, br)
      else:
        sh = _prop1(uh[None], idx_all, zeros128)
        h, c, uh = _cell_h(sx_all[t], ux[t], sh[0], uh, c, dis, wx, wh, br)
      outs.append(uh)
    ux = jnp.stack(outs)

  return (h[:N][None], c[:N][None])


# speedup vs baseline: 1.0637x; 1.0213x over previous
import functools

import jax
import jax.numpy as jnp
from jax import lax
from jax.experimental import pallas as pl
from jax.experimental.pallas import tpu as pltpu
from jax.experimental.pallas import tpu_sc as plsc

N = 10000
F = 128
H = 128
T = 4
NC = 2
NS = 16
NW = NC * NS
CHUNK = 128
NCHUNK = 80
NQ = 2
QC = NCHUNK // NQ
NBUF = 2
EW = CHUNK * NCHUNK
E_PAD = EW * NW
R = 632
N_PAD = R * NS
BLK = 1264
GRID = N_PAD // BLK

_MESH = plsc.VectorSubcoreMesh(
    core_axis_name="c", subcore_axis_name="s", num_cores=NC, num_subcores=NS)


HR = N_PAD // 16


def _hist_body(idx_hbm, out_hbm, idx_v, histv):
  c = lax.axis_index("c")
  s = lax.axis_index("s")
  ones = jnp.full((16,), 1.0, jnp.float32)

  def zero(i, carry):
    histv[i, :] = jnp.zeros((16,), jnp.float32)
    return carry

  lax.fori_loop(0, HR, zero, 0)
  for q in range(NQ):
    pltpu.sync_copy(idx_hbm.at[c, s, q], idx_v)

    def chunk(j, carry):
      def vec(k, carry2):
        idx16 = idx_v[QC + j, pl.ds(k * 16, 16)]
        row = lax.shift_right_logical(idx16, 4)
        col = lax.bitwise_and(idx16, 15)
        plsc.addupdate_scatter(histv, [row, col], ones)
        return carry2

      return lax.fori_loop(0, CHUNK // 16, vec, carry)

    lax.fori_loop(0, QC, chunk, 0)
  pltpu.sync_copy(histv, out_hbm.at[c, s])


_hist_call = pl.kernel(
    _hist_body,
    out_type=jax.ShapeDtypeStruct((NC, NS, HR, 16), jnp.float32),
    mesh=_MESH,
    compiler_params=pltpu.CompilerParams(needs_layout_passes=False),
    scratch_types=[
        pltpu.VMEM((2 * QC, CHUNK), jnp.int32),
        pltpu.VMEM((HR, 16), jnp.float32),
    ],
)


def _make_prop(nt):
  def body(table_hbm, idx_hbm, zeros_hbm, out_hbm,
           idx_v, buf0, buf1, acc, sg0, sg1):
    c = lax.axis_index("c")
    s = lax.axis_index("s")
    bufs = (buf0, buf1)
    sg = (sg0, sg1)
    for t in range(nt):
      table = table_hbm.at[t]
      pltpu.sync_copy(zeros_hbm.at[pl.ds(s * R, R)], acc.at[pl.ds(s * R, R)])
      plsc.subcore_barrier()
      for q in range(NQ):
        pltpu.sync_copy(idx_hbm.at[c, s, q], idx_v)
        pltpu.async_copy(table.at[idx_v.at[0]], buf0, sg0)
        pltpu.async_copy(table.at[idx_v.at[1]], buf1, sg1)

        def body2(j2, carry):
          for p in range(NBUF):
            j = j2 * NBUF + p
            pltpu.make_async_copy(table.at[idx_v.at[j]], bufs[p], sg[p]).wait()
            pltpu.sync_copy(bufs[p], acc.at[idx_v.at[QC + j]], add=True)

            @pl.when(j + 2 < QC)
            def _issue():
              pltpu.async_copy(table.at[idx_v.at[j + 2]], bufs[p], sg[p])
          return carry

        lax.fori_loop(0, QC // NBUF, body2, 0)
      plsc.subcore_barrier()
      pltpu.sync_copy(acc.at[pl.ds(s * R, R)], out_hbm.at[t, c, pl.ds(s * R, R)])

  return pl.kernel(
      body,
      out_type=jax.ShapeDtypeStruct((nt, NC, N_PAD, F), jnp.float32),
      mesh=_MESH,
      scratch_types=(
          [pltpu.VMEM((2 * QC, CHUNK), jnp.int32)]
          + [pltpu.VMEM((CHUNK, F), jnp.float32)] * NBUF
          + [pltpu.VMEM_SHARED((N_PAD, F), jnp.float32)]
          + [pltpu.SemaphoreType.DMA] * NBUF
      ),
  )


_prop1 = _make_prop(1)
_prop4 = _make_prop(T)


def _prep_body(hist_ref, x_ref, dis_ref, ux_ref):
  deg = jnp.sum(hist_ref[...], axis=1, keepdims=True) + 1.0
  d = lax.rsqrt(deg)
  dis_ref[...] = jnp.broadcast_to(d, (BLK, F))
  for t in range(T):
    ux_ref[t] = x_ref[t] * d


_prep_call = pl.pallas_call(
    _prep_body,
    grid=(GRID,),
    in_specs=[
        pl.BlockSpec((BLK, NC * NS), lambda i: (i, 0)),
        pl.BlockSpec((T, BLK, F), lambda i: (0, i, 0)),
    ],
    out_specs=[
        pl.BlockSpec((BLK, F), lambda i: (i, 0)),
        pl.BlockSpec((T, BLK, F), lambda i: (0, i, 0)),
    ],
    out_shape=[
        jax.ShapeDtypeStruct((N_PAD, F), jnp.float32),
        jax.ShapeDtypeStruct((T, N_PAD, F), jnp.float32),
    ],
)


def _make_cell(has_h):
  def body(*refs):
    if has_h:
      (sx_ref, ux_ref, sh_ref, uh_ref, c_ref, dis_ref, wx_ref, wh_ref, b_ref,
       h_o, c_o, uh_o) = refs
    else:
      (sx_ref, ux_ref, c_ref, dis_ref, wx_ref, b_ref, h_o, c_o, uh_o) = refs
    dis = dis_ref[...]
    px = dis * (sx_ref[0] + sx_ref[1] + ux_ref[...])
    cc = jnp.dot(px, wx_ref[...], preferred_element_type=jnp.float32)
    cc = cc + b_ref[...]
    if has_h:
      ph = dis * (sh_ref[0] + sh_ref[1] + uh_ref[...])
      cc = cc + jnp.dot(ph, wh_ref[...], preferred_element_type=jnp.float32)
    gi = jax.nn.sigmoid(cc[:, 0:H])
    gf = jax.nn.sigmoid(cc[:, H:2 * H])
    go = jax.nn.sigmoid(cc[:, 2 * H:3 * H])
    gg = jnp.tanh(cc[:, 3 * H:4 * H])
    c_new = gf * c_ref[...] + gi * gg
    h_new = go * jnp.tanh(c_new)
    h_o[...] = h_new
    c_o[...] = c_new
    uh_o[...] = dis * h_new

  part = pl.BlockSpec((NC, BLK, F), lambda i: (0, i, 0))
  full = pl.BlockSpec((BLK, F), lambda i: (i, 0))
  wspec = pl.BlockSpec((F, 4 * H), lambda i: (0, 0))
  bspec = pl.BlockSpec((1, 4 * H), lambda i: (0, 0))
  if has_h:
    in_specs = [part, full, part, full, full, full, wspec, wspec, bspec]
  else:
    in_specs = [part, full, full, full, wspec, bspec]
  return pl.pallas_call(
      body,
      grid=(GRID,),
      in_specs=in_specs,
      out_specs=[full, full, full],
      out_shape=[jax.ShapeDtypeStruct((N_PAD, F), jnp.float32)] * 3,
  )


_cell_h = _make_cell(True)
_cell_nh = _make_cell(False)


def kernel(x, edge_index, W0, b0, W1, b1):
  src = edge_index[0]
  dst = edge_index[1]
  pad = jnp.full((E_PAD - src.shape[0],), N, dtype=jnp.int32)
  src_q = jnp.concatenate([src, pad]).reshape(NC, NS, NQ, QC, CHUNK)
  dst_q = jnp.concatenate([dst, pad]).reshape(NC, NS, NQ, QC, CHUNK)
  idx_all = jnp.concatenate([src_q, dst_q], axis=3)

  zeros128 = jnp.zeros((N_PAD, F), jnp.float32)

  hist = jnp.transpose(_hist_call(idx_all).reshape(NC * NS, N_PAD))
  x_pad = jnp.pad(x[0], ((0, 0), (0, N_PAD - N), (0, 0)))
  dis, ux0 = _prep_call(hist, x_pad)

  b0r = b0.reshape(1, 4 * H)
  b1r = b1.reshape(1, 4 * H)
  params = [(W0[:F], W0[F:], b0r), (W1[:H], W1[H:], b1r)]

  ux = ux0
  h = c = None
  for layer in range(2):
    wx, wh, br = params[layer]
    sx_all = _prop4(ux, idx_all, zeros128)
    outs = []
    for t in range(T):
      if t == 0:
        c_prev = zeros128
        h, c, uh = _cell_nh(sx_all[t], ux[t], c_prev, dis, wx, br)
      else:
        sh = _prop1(uh[None], idx_all, zeros128)
        h, c, uh = _cell_h(sx_all[t], ux[t], sh[0], uh, c, dis, wx, wh, br)
      outs.append(uh)
    ux = jnp.stack(outs)

  return (h[:N][None], c[:N][None])
